# Initial kernel scaffold; baseline (speedup 1.0000x reference)
import jax
import jax.numpy as jnp
from jax.experimental import pallas as pl


def kernel(x, gap_edge_index, Wq, bq, Wk, bk, Wv, bv, Wo, bo):
    raise NotImplementedError("write your pallas kernel here")


# speedup vs baseline: 12.9487x; 12.9487x over previous
import functools

import jax
import jax.numpy as jnp
import numpy as np
from jax import lax
from jax.experimental import pallas as pl
from jax.experimental.pallas import tpu as pltpu
from jax.experimental.pallas import tpu_sc as plsc

N = 10000
E = 320000
H = 128
HEADS = 8
HD = 16
NW = 32
EPW = E // NW
C = 80
NCH = EPW // C
GPC = C // 16
NPAD = 10240
ROWS_PER_TILE = NPAD // 16
SBLK = HEADS * C

_mesh = plsc.VectorSubcoreMesh(core_axis_name="c", subcore_axis_name="s",
                               num_cores=2, num_subcores=16)


def _qkv_body(x_ref, wq_ref, bq_ref, wk_ref, bk_ref, wv_ref, bv_ref,
              q_ref, k_ref, v_ref):
    x = x_ref[...]
    dn = (((1,), (1,)), ((), ()))
    q = lax.dot_general(x, wq_ref[...], dn, preferred_element_type=jnp.float32)
    q_ref[...] = (q + bq_ref[...]) * (1.0 / (HD ** 0.5))
    k = lax.dot_general(x, wk_ref[...], dn, preferred_element_type=jnp.float32)
    k_ref[...] = k + bk_ref[...]
    v = lax.dot_general(x, wv_ref[...], dn, preferred_element_type=jnp.float32)
    v_ref[...] = v + bv_ref[...]


def _qkv(x, Wq, bq, Wk, bk, Wv, bv):
    nb = 10
    blk = N // nb
    row_spec = pl.BlockSpec((blk, H), lambda i: (i, 0))
    w_spec = pl.BlockSpec((H, H), lambda i: (0, 0))
    b_spec = pl.BlockSpec((1, H), lambda i: (0, 0))
    return pl.pallas_call(
        _qkv_body,
        grid=(nb,),
        in_specs=[row_spec, w_spec, b_spec, w_spec, b_spec, w_spec, b_spec],
        out_specs=[row_spec, row_spec, row_spec],
        out_shape=[jax.ShapeDtypeStruct((N, H), jnp.float32)] * 3,
    )(x, Wq, bq.reshape(1, H), Wk, bk.reshape(1, H), Wv, bv.reshape(1, H))


_A_SCRATCH = (
    [pltpu.VMEM((EPW,), jnp.int32),
     pltpu.VMEM((EPW,), jnp.int32)]
    + [pltpu.VMEM((C, H), jnp.float32) for _ in range(5)]
    + [pltpu.VMEM((C, H), jnp.float32) for _ in range(5)]
    + [pltpu.VMEM((SBLK,), jnp.float32) for _ in range(5)]
    + [pltpu.VMEM((HEADS * 16,), jnp.float32)]
    + [pltpu.SemaphoreType.DMA for _ in range(5)]
    + [pltpu.SemaphoreType.DMA for _ in range(5)]
    + [pltpu.SemaphoreType.DMA]
)


@functools.partial(
    pl.kernel,
    out_type=[
        jax.ShapeDtypeStruct((NW * HEADS * EPW,), jnp.float32),
        jax.ShapeDtypeStruct((NW * HEADS * 16,), jnp.float32),
    ],
    mesh=_mesh,
    compiler_params=pltpu.CompilerParams(needs_layout_passes=False),
    scratch_types=_A_SCRATCH,
)
def _edge_scores(q_hbm, k_hbm, dst_hbm, src_hbm, s_hbm, mx_hbm,
                 dstf, srcf,
                 qd0, qd1, qd2, qd3, qd4,
                 ks0, ks1, ks2, ks3, ks4,
                 sb0, sb1, sb2, sb3, sb4,
                 mbuf,
                 sq0, sq1, sq2, sq3, sq4,
                 sk0, sk1, sk2, sk3, sk4,
                 semw):
    cid = lax.axis_index("c")
    sid = lax.axis_index("s")
    wid = cid * 16 + sid
    lanes = lax.iota(jnp.int32, 16)
    qb = [qd0, qd1, qd2, qd3, qd4]
    kb = [ks0, ks1, ks2, ks3, ks4]
    sb = [sb0, sb1, sb2, sb3, sb4]
    sq = [sq0, sq1, sq2, sq3, sq4]
    sk = [sk0, sk1, sk2, sk3, sk4]

    ebase = wid * EPW
    pltpu.sync_copy(dst_hbm.at[pl.ds(ebase, EPW)], dstf)
    pltpu.sync_copy(src_hbm.at[pl.ds(ebase, EPW)], srcf)

    sobase = wid * (HEADS * EPW)
    neg = jnp.full((16,), -3.0e38, jnp.float32)
    m_init = tuple(neg for _ in range(HEADS))

    def body(jj, m_carry):
        j0 = jj * 5
        cq = []
        ck = []
        for i in range(5):
            off = (j0 + i) * C
            cq.append(pltpu.async_copy(q_hbm.at[dstf.at[pl.ds(off, C)]],
                                       qb[i], sq[i]))
            ck.append(pltpu.async_copy(k_hbm.at[srcf.at[pl.ds(off, C)]],
                                       kb[i], sk[i]))
        cw = []
        for i in range(5):
            cq[i].wait()
            ck[i].wait()
            qd = qb[i]
            ks = kb[i]
            sbuf = sb[i]

            def group_body(g, m, qd=qd, ks=ks, sbuf=sbuf):
                e0 = g * 16
                rows = e0 + lanes
                mnew = []
                for h in range(HEADS):
                    acc = jnp.zeros((16,), jnp.float32)
                    for d in range(HD):
                        col = jnp.full((16,), h * HD + d, jnp.int32)
                        qv = plsc.load_gather(qd, [rows, col])
                        kv = plsc.load_gather(ks, [rows, col])
                        acc = acc + qv * kv
                    sbuf[pl.ds(h * C + e0, 16)] = acc
                    mnew.append(jnp.maximum(m[h], acc))
                return tuple(mnew)

            m_carry = lax.fori_loop(0, GPC, group_body, m_carry)
            cw.append(pltpu.async_copy(
                sbuf, s_hbm.at[pl.ds(sobase + (j0 + i) * SBLK, SBLK)], semw))
        for i in range(5):
            cw[i].wait()
        return m_carry

    m_fin = lax.fori_loop(0, NCH // 5, body, m_init)
    for h in range(HEADS):
        mbuf[pl.ds(h * 16, 16)] = m_fin[h]
    pltpu.sync_copy(mbuf, mx_hbm.at[pl.ds(wid * (HEADS * 16), HEADS * 16)])


_B_SCRATCH = (
    [pltpu.VMEM((C,), jnp.int32),
     pltpu.VMEM((C,), jnp.int32),
     pltpu.VMEM((2 * C,), jnp.int32),
     pltpu.VMEM((C, H), jnp.float32),
     pltpu.VMEM((C, H), jnp.float32),
     pltpu.VMEM((C, H), jnp.float32),
     pltpu.VMEM((C, H), jnp.float32),
     pltpu.VMEM((SBLK,), jnp.float32),
     pltpu.VMEM((SBLK,), jnp.float32),
     pltpu.VMEM((1024,), jnp.float32),
     pltpu.VMEM((HEADS * 16,), jnp.float32),
     pltpu.VMEM_SHARED((NPAD, H), jnp.float32),
     pltpu.SemaphoreType.DMA,
     pltpu.SemaphoreType.DMA,
     pltpu.SemaphoreType.DMA,
     pltpu.SemaphoreType.DMA,
     pltpu.SemaphoreType.DMA,
     pltpu.SemaphoreType.DMA]
)


@functools.partial(
    pl.kernel,
    out_type=[
        jax.ShapeDtypeStruct((2, NPAD, H), jnp.float32),
        jax.ShapeDtypeStruct((NW * HEADS * 16,), jnp.float32),
    ],
    mesh=_mesh,
    compiler_params=pltpu.CompilerParams(needs_layout_passes=False),
    scratch_types=_B_SCRATCH,
)
def _aggregate(s_hbm, mx_hbm, v_hbm, dst_hbm, src_hbm, z0_hbm,
               att_hbm, zp_hbm,
               dva, dvb, sv, vb0, vb1, ob0, ob1, pb0, pb1, mxbuf, zbuf, acc,
               semi, semv0, semv1, sems0, sems1, semsc):
    cid = lax.axis_index("c")
    sid = lax.axis_index("s")
    wid = cid * 16 + sid
    lanes = lax.iota(jnp.int32, 16)
    dv = [dva, dvb]
    vb = [vb0, vb1]
    ob = [ob0, ob1]
    pb = [pb0, pb1]
    semv = [semv0, semv1]
    sems = [sems0, sems1]

    msplat = []
    mvec = [jnp.full((16,), -3.0e38, jnp.float32) for _ in range(HEADS)]
    for part in range(4):
        pltpu.sync_copy(mx_hbm.at[pl.ds(part * 1024, 1024)], mxbuf)
        for w in range(8):
            for h in range(HEADS):
                mvec[h] = jnp.maximum(
                    mvec[h], mxbuf[pl.ds(w * (HEADS * 16) + h * 16, 16)])
    for h in range(HEADS):
        msplat.append(jnp.full((16,), jnp.max(mvec[h]), jnp.float32))

    pltpu.sync_copy(z0_hbm,
                    acc.at[pl.ds(sid * ROWS_PER_TILE, ROWS_PER_TILE)])
    plsc.subcore_barrier()

    ebase = wid * EPW
    sobase = wid * (HEADS * EPW)

    def load_idx(body_idx):
        off = ebase + body_idx * (2 * C)
        pltpu.async_copy(dst_hbm.at[pl.ds(off, C)], dva, semi)
        pltpu.async_copy(dst_hbm.at[pl.ds(off + C, C)], dvb, semi)
        pltpu.async_copy(src_hbm.at[pl.ds(off, 2 * C)], sv, semi)

    def drain_idx():
        pltpu.make_async_copy(dst_hbm.at[pl.ds(0, C)], dva, semi).wait()
        pltpu.make_async_copy(dst_hbm.at[pl.ds(0, C)], dvb, semi).wait()
        pltpu.make_async_copy(src_hbm.at[pl.ds(0, 2 * C)], sv, semi).wait()

    load_idx(0)
    drain_idx()

    NB = NCH // 2

    def compute_chunk(i, z_carry):
        obuf = ob[i]
        vbuf = vb[i]
        pbuf = pb[i]

        def group_body(g, z, obuf=obuf, vbuf=vbuf, pbuf=pbuf):
            e0 = g * 16
            rows = e0 + lanes
            znew = []
            for h in range(HEADS):
                svv = pbuf[pl.ds(h * C + e0, 16)]
                p = jnp.exp(svv - msplat[h])
                znew.append(z[h] + p)
                for d in range(HD):
                    col = jnp.full((16,), h * HD + d, jnp.int32)
                    vv = plsc.load_gather(vbuf, [rows, col])
                    plsc.store_scatter(obuf, [rows, col], vv * p)
            return tuple(znew)

        return lax.fori_loop(0, GPC, group_body, z_carry)

    z_init = tuple(jnp.zeros((16,), jnp.float32) for _ in range(HEADS))

    def body(jj, z_carry):
        j0 = jj * 2
        cv = []
        cs = []
        for i in range(2):
            cv.append(pltpu.async_copy(
                v_hbm.at[sv.at[pl.ds(i * C, C)]], vb[i], semv[i]))
            cs.append(pltpu.async_copy(
                s_hbm.at[pl.ds(sobase + (j0 + i) * SBLK, SBLK)],
                pb[i], sems[i]))
        csc = []
        for i in range(2):
            cv[i].wait()
            cs[i].wait()
            z_carry = compute_chunk(i, z_carry)
            csc.append(pltpu.async_copy(ob[i], acc.at[dv[i]], semsc,
                                        add=True))
        csc[0].wait()
        csc[1].wait()

        @pl.when(jj + 1 < NB)
        def _():
            load_idx(jj + 1)
            drain_idx()
        return z_carry

    z_fin = lax.fori_loop(0, NB, body, z_init)

    jlast = NCH - 1
    off = ebase + jlast * C
    pltpu.sync_copy(dst_hbm.at[pl.ds(off, C)], dva)
    pltpu.sync_copy(src_hbm.at[pl.ds(off, C)], sv.at[pl.ds(0, C)])
    cv = pltpu.async_copy(v_hbm.at[sv.at[pl.ds(0, C)]], vb0, semv0)
    pltpu.sync_copy(s_hbm.at[pl.ds(sobase + jlast * SBLK, SBLK)], pb0)
    cv.wait()
    z_fin = compute_chunk(0, z_fin)
    pltpu.async_copy(ob0, acc.at[dva], semsc, add=True).wait()

    for h in range(HEADS):
        zbuf[pl.ds(h * 16, 16)] = z_fin[h]
    pltpu.sync_copy(zbuf, zp_hbm.at[pl.ds(wid * (HEADS * 16), HEADS * 16)])
    plsc.subcore_barrier()
    pltpu.sync_copy(acc.at[pl.ds(sid * ROWS_PER_TILE, ROWS_PER_TILE)],
                    att_hbm.at[cid, pl.ds(sid * ROWS_PER_TILE, ROWS_PER_TILE)])


def _out_body(a0_ref, a1_ref, zp_ref, x_ref, wo_ref, bo_ref, ones_ref,
              o_ref):
    z = jnp.sum(zp_ref[...], axis=0, keepdims=True)
    zrow = lax.dot_general(z, ones_ref[...], (((1,), (0,)), ((), ())),
                           preferred_element_type=jnp.float32)
    att = (a0_ref[...] + a1_ref[...]) / zrow
    y = lax.dot_general(att, wo_ref[...], (((1,), (1,)), ((), ())),
                        preferred_element_type=jnp.float32)
    o_ref[...] = y + x_ref[...] + bo_ref[...]


def _output(a0, a1, zp, x, Wo, bo, ones_blk):
    nb = 10
    blk = N // nb
    row_spec = pl.BlockSpec((blk, H), lambda i: (i, 0))
    full_spec = pl.BlockSpec((H, H), lambda i: (0, 0))
    zp_spec = pl.BlockSpec((NW, H), lambda i: (0, 0))
    b_spec = pl.BlockSpec((1, H), lambda i: (0, 0))
    return pl.pallas_call(
        _out_body,
        grid=(nb,),
        in_specs=[row_spec, row_spec, zp_spec, row_spec, full_spec, b_spec,
                  full_spec],
        out_specs=row_spec,
        out_shape=jax.ShapeDtypeStruct((N, H), jnp.float32),
    )(a0, a1, zp, x, Wo, bo.reshape(1, H), ones_blk)


_ONES_BLK = np.kron(np.eye(HEADS, dtype=np.float32),
                    np.ones((HD, HD), dtype=np.float32))


def kernel(x, gap_edge_index, Wq, bq, Wk, bk, Wv, bv, Wo, bo):
    src = gap_edge_index[0]
    dst = gap_edge_index[1]
    q4, k, v = _qkv(x, Wq, bq, Wk, bk, Wv, bv)
    s, mx = _edge_scores(q4, k, dst, src)
    z0 = jnp.zeros((ROWS_PER_TILE, H), jnp.float32)
    att, zp = _aggregate(s, mx, v, dst, src, z0)
    ones_blk = jnp.asarray(_ONES_BLK)
    return _output(att[0, :N], att[1, :N], zp.reshape(NW, HEADS * 16),
                   x, Wo, bo, ones_blk)


# speedup vs baseline: 72.4290x; 5.5935x over previous
import functools

import jax
import jax.numpy as jnp
import numpy as np
from jax import lax
from jax.experimental import pallas as pl
from jax.experimental.pallas import tpu as pltpu
from jax.experimental.pallas import tpu_sc as plsc

N = 10000
E = 320000
H = 128
HEADS = 8
HD = 16
NW = 32
EPW = E // NW
C = 80
NCH = EPW // C
GPC = C // 16
NPAD = 10240
ROWS_PER_TILE = NPAD // 16
SBLK = HEADS * C

_mesh = plsc.VectorSubcoreMesh(core_axis_name="c", subcore_axis_name="s",
                               num_cores=2, num_subcores=16)


def _qkv_body(x_ref, wq_ref, bq_ref, wk_ref, bk_ref, wv_ref, bv_ref,
              q_ref, k_ref, v_ref):
    x = x_ref[...]
    dn = (((1,), (1,)), ((), ()))
    q = lax.dot_general(x, wq_ref[...], dn, preferred_element_type=jnp.float32)
    q_ref[...] = (q + bq_ref[...]) * (1.0 / (HD ** 0.5))
    k = lax.dot_general(x, wk_ref[...], dn, preferred_element_type=jnp.float32)
    k_ref[...] = k + bk_ref[...]
    v = lax.dot_general(x, wv_ref[...], dn, preferred_element_type=jnp.float32)
    v_ref[...] = v + bv_ref[...]


def _qkv(x, Wq, bq, Wk, bk, Wv, bv):
    nb = 10
    blk = N // nb
    row_spec = pl.BlockSpec((blk, H), lambda i: (i, 0))
    w_spec = pl.BlockSpec((H, H), lambda i: (0, 0))
    b_spec = pl.BlockSpec((1, H), lambda i: (0, 0))
    return pl.pallas_call(
        _qkv_body,
        grid=(nb,),
        in_specs=[row_spec, w_spec, b_spec, w_spec, b_spec, w_spec, b_spec],
        out_specs=[row_spec, row_spec, row_spec],
        out_shape=[jax.ShapeDtypeStruct((N, H), jnp.float32)] * 3,
    )(x, Wq, bq.reshape(1, H), Wk, bk.reshape(1, H), Wv, bv.reshape(1, H))


_A_SCRATCH = (
    [pltpu.VMEM((EPW,), jnp.int32),
     pltpu.VMEM((EPW,), jnp.int32)]
    + [pltpu.VMEM((C, H), jnp.float32) for _ in range(5)]
    + [pltpu.VMEM((C, H), jnp.float32) for _ in range(5)]
    + [pltpu.VMEM((SBLK,), jnp.float32) for _ in range(5)]
    + [pltpu.VMEM((HEADS * 16,), jnp.float32)]
    + [pltpu.SemaphoreType.DMA for _ in range(5)]
    + [pltpu.SemaphoreType.DMA for _ in range(5)]
    + [pltpu.SemaphoreType.DMA]
)


@functools.partial(
    pl.kernel,
    out_type=[
        jax.ShapeDtypeStruct((NW * HEADS * EPW,), jnp.float32),
        jax.ShapeDtypeStruct((NW * HEADS * 16,), jnp.float32),
    ],
    mesh=_mesh,
    compiler_params=pltpu.CompilerParams(needs_layout_passes=False),
    scratch_types=_A_SCRATCH,
)
def _edge_scores(q_hbm, k_hbm, dst_hbm, src_hbm, s_hbm, mx_hbm,
                 dstf, srcf,
                 qd0, qd1, qd2, qd3, qd4,
                 ks0, ks1, ks2, ks3, ks4,
                 sb0, sb1, sb2, sb3, sb4,
                 mbuf,
                 sq0, sq1, sq2, sq3, sq4,
                 sk0, sk1, sk2, sk3, sk4,
                 semw):
    cid = lax.axis_index("c")
    sid = lax.axis_index("s")
    wid = cid * 16 + sid
    lanes = lax.iota(jnp.int32, 16)
    qb = [qd0, qd1, qd2, qd3, qd4]
    kb = [ks0, ks1, ks2, ks3, ks4]
    sb = [sb0, sb1, sb2, sb3, sb4]
    sq = [sq0, sq1, sq2, sq3, sq4]
    sk = [sk0, sk1, sk2, sk3, sk4]

    ebase = wid * EPW
    pltpu.sync_copy(dst_hbm.at[pl.ds(ebase, EPW)], dstf)
    pltpu.sync_copy(src_hbm.at[pl.ds(ebase, EPW)], srcf)

    sobase = wid * (HEADS * EPW)
    neg = jnp.full((16,), -3.0e38, jnp.float32)
    m_init = tuple(neg for _ in range(HEADS))

    def body(jj, m_carry):
        j0 = jj * 5
        cq = []
        ck = []
        for i in range(5):
            off = (j0 + i) * C
            cq.append(pltpu.async_copy(q_hbm.at[dstf.at[pl.ds(off, C)]],
                                       qb[i], sq[i]))
            ck.append(pltpu.async_copy(k_hbm.at[srcf.at[pl.ds(off, C)]],
                                       kb[i], sk[i]))
        cw = []
        for i in range(5):
            cq[i].wait()
            ck[i].wait()
            qd = qb[i]
            ks = kb[i]
            sbuf = sb[i]

            def group_body(g, carry, qd=qd, ks=ks, sbuf=sbuf):
                e0 = g * 16
                for h in range(HEADS):
                    vec = jnp.zeros((16,), jnp.float32)
                    for ee in range(16):
                        e = e0 + ee
                        qrow = qd[e, pl.ds(h * HD, HD)]
                        krow = ks[e, pl.ds(h * HD, HD)]
                        s_sc = jnp.sum(qrow * krow)
                        vec = jnp.where(lanes == ee, s_sc, vec)
                    sbuf[pl.ds(h * C + e0, 16)] = vec
                return carry

            lax.fori_loop(0, GPC, group_body, 0)
            mnew = []
            for h in range(HEADS):
                mv = m_carry[h]
                for g in range(GPC):
                    mv = jnp.maximum(mv, sbuf[pl.ds(h * C + g * 16, 16)])
                mnew.append(mv)
            m_carry = tuple(mnew)
            cw.append(pltpu.async_copy(
                sbuf, s_hbm.at[pl.ds(sobase + (j0 + i) * SBLK, SBLK)], semw))
        for i in range(5):
            cw[i].wait()
        return m_carry

    m_fin = lax.fori_loop(0, NCH // 5, body, m_init)
    for h in range(HEADS):
        mbuf[pl.ds(h * 16, 16)] = m_fin[h]
    pltpu.sync_copy(mbuf, mx_hbm.at[pl.ds(wid * (HEADS * 16), HEADS * 16)])


_B_SCRATCH = (
    [pltpu.VMEM((C,), jnp.int32),
     pltpu.VMEM((C,), jnp.int32),
     pltpu.VMEM((2 * C,), jnp.int32),
     pltpu.VMEM((C, H), jnp.float32),
     pltpu.VMEM((C, H), jnp.float32),
     pltpu.VMEM((C, H), jnp.float32),
     pltpu.VMEM((C, H), jnp.float32),
     pltpu.VMEM((SBLK,), jnp.float32),
     pltpu.VMEM((SBLK,), jnp.float32),
     pltpu.VMEM((1024,), jnp.float32),
     pltpu.VMEM((HEADS * 16,), jnp.float32),
     pltpu.VMEM_SHARED((NPAD, H), jnp.float32),
     pltpu.SemaphoreType.DMA,
     pltpu.SemaphoreType.DMA,
     pltpu.SemaphoreType.DMA,
     pltpu.SemaphoreType.DMA,
     pltpu.SemaphoreType.DMA,
     pltpu.SemaphoreType.DMA]
)


@functools.partial(
    pl.kernel,
    out_type=[
        jax.ShapeDtypeStruct((2, NPAD, H), jnp.float32),
        jax.ShapeDtypeStruct((NW * HEADS * 16,), jnp.float32),
    ],
    mesh=_mesh,
    compiler_params=pltpu.CompilerParams(needs_layout_passes=False),
    scratch_types=_B_SCRATCH,
)
def _aggregate(s_hbm, mx_hbm, v_hbm, dst_hbm, src_hbm, z0_hbm,
               att_hbm, zp_hbm,
               dva, dvb, sv, vb0, vb1, ob0, ob1, pb0, pb1, mxbuf, zbuf, acc,
               semi, semv0, semv1, sems0, sems1, semsc):
    cid = lax.axis_index("c")
    sid = lax.axis_index("s")
    wid = cid * 16 + sid
    lanes = lax.iota(jnp.int32, 16)
    dv = [dva, dvb]
    vb = [vb0, vb1]
    ob = [ob0, ob1]
    pb = [pb0, pb1]
    semv = [semv0, semv1]
    sems = [sems0, sems1]

    msplat = []
    mvec = [jnp.full((16,), -3.0e38, jnp.float32) for _ in range(HEADS)]
    for part in range(4):
        pltpu.sync_copy(mx_hbm.at[pl.ds(part * 1024, 1024)], mxbuf)
        for w in range(8):
            for h in range(HEADS):
                mvec[h] = jnp.maximum(
                    mvec[h], mxbuf[pl.ds(w * (HEADS * 16) + h * 16, 16)])
    for h in range(HEADS):
        msplat.append(jnp.full((16,), jnp.max(mvec[h]), jnp.float32))

    pltpu.sync_copy(z0_hbm,
                    acc.at[pl.ds(sid * ROWS_PER_TILE, ROWS_PER_TILE)])
    plsc.subcore_barrier()

    ebase = wid * EPW
    sobase = wid * (HEADS * EPW)

    def load_idx(body_idx):
        off = ebase + body_idx * (2 * C)
        pltpu.async_copy(dst_hbm.at[pl.ds(off, C)], dva, semi)
        pltpu.async_copy(dst_hbm.at[pl.ds(off + C, C)], dvb, semi)
        pltpu.async_copy(src_hbm.at[pl.ds(off, 2 * C)], sv, semi)

    def drain_idx():
        pltpu.make_async_copy(dst_hbm.at[pl.ds(0, C)], dva, semi).wait()
        pltpu.make_async_copy(dst_hbm.at[pl.ds(0, C)], dvb, semi).wait()
        pltpu.make_async_copy(src_hbm.at[pl.ds(0, 2 * C)], sv, semi).wait()

    load_idx(0)
    drain_idx()

    NB = NCH // 2

    def compute_chunk(i, z_carry):
        obuf = ob[i]
        vbuf = vb[i]
        pbuf = pb[i]

        def group_body(g, z, obuf=obuf, vbuf=vbuf, pbuf=pbuf):
            e0 = g * 16
            znew = []
            pv = []
            for h in range(HEADS):
                svv = pbuf[pl.ds(h * C + e0, 16)]
                p = jnp.exp(svv - msplat[h])
                znew.append(z[h] + p)
                pv.append(p)
            for ee in range(16):
                e = e0 + ee
                for h in range(HEADS):
                    ps = pv[h][ee]
                    vrow = vbuf[e, pl.ds(h * HD, HD)]
                    obuf[e, pl.ds(h * HD, HD)] = vrow * ps
            return tuple(znew)

        return lax.fori_loop(0, GPC, group_body, z_carry)

    z_init = tuple(jnp.zeros((16,), jnp.float32) for _ in range(HEADS))

    def body(jj, z_carry):
        j0 = jj * 2
        cv = []
        cs = []
        for i in range(2):
            cv.append(pltpu.async_copy(
                v_hbm.at[sv.at[pl.ds(i * C, C)]], vb[i], semv[i]))
            cs.append(pltpu.async_copy(
                s_hbm.at[pl.ds(sobase + (j0 + i) * SBLK, SBLK)],
                pb[i], sems[i]))
        csc = []
        for i in range(2):
            cv[i].wait()
            cs[i].wait()
            z_carry = compute_chunk(i, z_carry)
            csc.append(pltpu.async_copy(ob[i], acc.at[dv[i]], semsc,
                                        add=True))
        csc[0].wait()
        csc[1].wait()

        @pl.when(jj + 1 < NB)
        def _():
            load_idx(jj + 1)
            drain_idx()
        return z_carry

    z_fin = lax.fori_loop(0, NB, body, z_init)

    jlast = NCH - 1
    off = ebase + jlast * C
    pltpu.sync_copy(dst_hbm.at[pl.ds(off, C)], dva)
    pltpu.sync_copy(src_hbm.at[pl.ds(off, C)], sv.at[pl.ds(0, C)])
    cv = pltpu.async_copy(v_hbm.at[sv.at[pl.ds(0, C)]], vb0, semv0)
    pltpu.sync_copy(s_hbm.at[pl.ds(sobase + jlast * SBLK, SBLK)], pb0)
    cv.wait()
    z_fin = compute_chunk(0, z_fin)
    pltpu.async_copy(ob0, acc.at[dva], semsc, add=True).wait()

    for h in range(HEADS):
        zbuf[pl.ds(h * 16, 16)] = z_fin[h]
    pltpu.sync_copy(zbuf, zp_hbm.at[pl.ds(wid * (HEADS * 16), HEADS * 16)])
    plsc.subcore_barrier()
    pltpu.sync_copy(acc.at[pl.ds(sid * ROWS_PER_TILE, ROWS_PER_TILE)],
                    att_hbm.at[cid, pl.ds(sid * ROWS_PER_TILE, ROWS_PER_TILE)])


def _out_body(a0_ref, a1_ref, zp_ref, x_ref, wo_ref, bo_ref, ones_ref,
              o_ref):
    z = jnp.sum(zp_ref[...], axis=0, keepdims=True)
    zrow = lax.dot_general(z, ones_ref[...], (((1,), (0,)), ((), ())),
                           preferred_element_type=jnp.float32)
    att = (a0_ref[...] + a1_ref[...]) / zrow
    y = lax.dot_general(att, wo_ref[...], (((1,), (1,)), ((), ())),
                        preferred_element_type=jnp.float32)
    o_ref[...] = y + x_ref[...] + bo_ref[...]


def _output(a0, a1, zp, x, Wo, bo, ones_blk):
    nb = 10
    blk = N // nb
    row_spec = pl.BlockSpec((blk, H), lambda i: (i, 0))
    full_spec = pl.BlockSpec((H, H), lambda i: (0, 0))
    zp_spec = pl.BlockSpec((NW, H), lambda i: (0, 0))
    b_spec = pl.BlockSpec((1, H), lambda i: (0, 0))
    return pl.pallas_call(
        _out_body,
        grid=(nb,),
        in_specs=[row_spec, row_spec, zp_spec, row_spec, full_spec, b_spec,
                  full_spec],
        out_specs=row_spec,
        out_shape=jax.ShapeDtypeStruct((N, H), jnp.float32),
    )(a0, a1, zp, x, Wo, bo.reshape(1, H), ones_blk)


_ONES_BLK = np.kron(np.eye(HEADS, dtype=np.float32),
                    np.ones((HD, HD), dtype=np.float32))


def kernel(x, gap_edge_index, Wq, bq, Wk, bk, Wv, bv, Wo, bo):
    src = gap_edge_index[0]
    dst = gap_edge_index[1]
    q4, k, v = _qkv(x, Wq, bq, Wk, bk, Wv, bv)
    s, mx = _edge_scores(q4, k, dst, src)
    z0 = jnp.zeros((ROWS_PER_TILE, H), jnp.float32)
    att, zp = _aggregate(s, mx, v, dst, src, z0)
    ones_blk = jnp.asarray(_ONES_BLK)
    return _output(att[0, :N], att[1, :N], zp.reshape(NW, HEADS * 16),
                   x, Wo, bo, ones_blk)
